# baseline (speedup 1.0000x reference)
import functools

import jax
import jax.numpy as jnp
from jax.experimental import pallas as pl


def _gru_layer(x_seq, p):
    H = p['whh'].shape[1]
    def step(h, x_t):
        gi = x_t @ p['wih'].T + p['bih']
        gh = h @ p['whh'].T + p['bhh']
        ir, iz, i_n = jnp.split(gi, 3, axis=-1)
        hr, hz, hn = jnp.split(gh, 3, axis=-1)
        r = jax.nn.sigmoid(ir + hr)
        z = jax.nn.sigmoid(iz + hz)
        n = jnp.tanh(i_n + r * hn)
        h_new = (1.0 - z) * n + z * h
        return h_new, h_new
    h0 = jnp.zeros((x_seq.shape[1], H), x_seq.dtype)
    _, ys = jax.lax.scan(step, h0, x_seq)
    return ys


def _batchnorm(x, p):
    mu = jnp.mean(x, axis=0)
    var = jnp.var(x, axis=0)
    return p['g'] * (x - mu) / jnp.sqrt(var + 1e-5) + p['b']


def _gatv2(x, src, dst, p, heads):
    N = x.shape[0]
    O = p['att'].shape[1]
    src = jnp.concatenate([src, jnp.arange(N, dtype=src.dtype)])
    dst = jnp.concatenate([dst, jnp.arange(N, dtype=dst.dtype)])
    xl = (x @ p['wl'].T).reshape(N, heads, O)
    xr = (x @ p['wr'].T).reshape(N, heads, O)
    xj = xl[src]
    xi = xr[dst]
    e = jax.nn.leaky_relu(xi + xj, 0.2)
    logits = jnp.sum(e * p['att'][None, :, :], axis=-1)
    m = jax.ops.segment_max(logits, dst, num_segments=N)
    ex = jnp.exp(logits - m[dst])
    denom = jax.ops.segment_sum(ex, dst, num_segments=N)
    alpha = ex / (denom[dst] + 1e-16)
    msg = xj * alpha[:, :, None]
    summed = jax.ops.segment_sum(msg, dst, num_segments=N)
    cnt = jax.ops.segment_sum(jnp.ones((dst.shape[0],), x.dtype), dst, num_segments=N)
    out = summed / jnp.maximum(cnt, 1.0)[:, None, None]
    return out.reshape(N, heads * O) + p['bias']


def _pred_body(x_ref, w1_ref, b1_ref, w2_ref, b2_ref, w3_ref, b3_ref, o_ref):
    x = x_ref[...]
    h1 = jnp.maximum(x @ w1_ref[...].T + b1_ref[...], 0.0)
    h2 = jnp.maximum(h1 @ w2_ref[...].T + b2_ref[...], 0.0)
    o_ref[...] = h2 @ w3_ref[...].T + b3_ref[...]


def _pred_mlp(x, p):
    N = x.shape[0]
    B = 2048
    grid = (pl.cdiv(N, B),)
    w3p = jnp.zeros((128, 32), x.dtype).at[0].set(p['w3'][0])
    b3p = jnp.zeros((128,), x.dtype).at[0].set(p['b3'][0])
    full = lambda s: pl.BlockSpec(s, lambda i: (0,) * len(s))
    out = pl.pallas_call(
        _pred_body,
        grid=grid,
        in_specs=[pl.BlockSpec((B, x.shape[1]), lambda i: (i, 0)),
                  full(p['w1'].shape), full(p['b1'].shape),
                  full(p['w2'].shape), full(p['b2'].shape),
                  full(w3p.shape), full(b3p.shape)],
        out_specs=pl.BlockSpec((B, 128), lambda i: (i, 0)),
        out_shape=jax.ShapeDtypeStruct((N, 128), x.dtype),
    )(x, p['w1'], p['b1'], p['w2'], p['b2'], w3p, b3p)
    return out[:, :1]


def kernel(x_low, x_high, edge_index_l2h, edge_index_hh, params):
    xs = jnp.transpose(x_low, (1, 0, 2))
    xs = _gru_layer(xs, params['gru'][0])
    xs = _gru_layer(xs, params['gru'][1])
    enc = jnp.transpose(xs, (1, 0, 2)).reshape(x_low.shape[0], -1)
    enc = jax.nn.relu(enc @ params['dense']['w'].T + params['dense']['b'])
    src, dst = edge_index_l2h[0], edge_index_l2h[1]
    msg = enc[src]
    agg = jax.ops.segment_sum(msg, dst, num_segments=x_high.shape[0])
    cnt = jax.ops.segment_sum(jnp.ones((dst.shape[0],), enc.dtype), dst,
                              num_segments=x_high.shape[0])
    agg = agg / jnp.maximum(cnt, 1.0)[:, None]
    x = (agg @ params['down']['w_rel'].T + params['down']['b_rel']
         + x_high @ params['down']['w_root'].T)
    x = _batchnorm(x, params['bn'][0])
    heads = [2, 2, 2, 2, 1]
    for i in range(4):
        x = _gatv2(x, edge_index_hh[0], edge_index_hh[1], params['gat'][i], heads[i])
        x = jax.nn.relu(_batchnorm(x, params['bn'][i + 1]))
    x = jax.nn.relu(_gatv2(x, edge_index_hh[0], edge_index_hh[1], params['gat'][4], 1))
    return _pred_mlp(x, params['pred'])


# speedup vs baseline: 7.0698x; 7.0698x over previous
import functools

import jax
import jax.numpy as jnp
from jax import lax
from jax.experimental import pallas as pl
from jax.experimental.pallas import tpu as pltpu
from jax.experimental.pallas import tpu_sc as plsc

N = 50000
CHK = 784
NJOB = 64
NPAD = CHK * NJOB
EBLK = 64
NW = 32
BN_EPS = 1e-5


def _lane_take(x, idx):
    dn = lax.GatherDimensionNumbers(offset_dims=(), collapsed_slice_dims=(0,),
                                    start_index_map=(0,))
    return lax.gather(x, idx[:, None], dn, (1,),
                      mode=lax.GatherScatterMode.PROMISE_IN_BOUNDS)


def _runs(dl, iota):
    prev = _lane_take(dl, jnp.maximum(iota - 1, 0))
    nxt = _lane_take(dl, jnp.minimum(iota + 1, 15))
    frst = (iota == 0) | (dl != prev)
    last = (iota == 15) | (dl != nxt)
    fidx = plsc.cummax(
        jnp.where(frst, iota.astype(jnp.float32), 0.0)).astype(jnp.int32)
    pidx = jnp.maximum(fidx - 1, 0)
    return last, pidx, fidx > 0


def _segtot(vals, pidx, haveprev):
    c = plsc.cumsum(vals)
    cp = _lane_take(c, pidx)
    cp = jnp.where(haveprev, cp, 0.0)
    return c - cp


def _make_sc_seg(E, F, H, with_att):
    mesh = plsc.VectorSubcoreMesh(core_axis_name="c", subcore_axis_name="s")
    rounds = (H * NJOB) // NW

    if with_att:
        out_type = [jax.ShapeDtypeStruct((H * NPAD, F), jnp.float32),
                    jax.ShapeDtypeStruct((H * NPAD,), jnp.float32)]
    elif F > 0:
        out_type = [jax.ShapeDtypeStruct((NPAD, F), jnp.float32)]
    else:
        out_type = [jax.ShapeDtypeStruct((NPAD,), jnp.float32)]

    scratch = []
    if F > 0:
        scratch.append(pltpu.VMEM((CHK, F), jnp.float32))
    if with_att or F == 0:
        scratch.append(pltpu.VMEM((CHK,), jnp.float32))
    if F > 0:
        scratch.append(pltpu.VMEM((EBLK,), jnp.int32))
        scratch.append(pltpu.VMEM((EBLK,), jnp.int32))
        scratch.append(pltpu.VMEM((EBLK, 128), jnp.float32))
    scratch.append(pltpu.VMEM((EBLK,), jnp.int32))
    if with_att:
        scratch.append(pltpu.VMEM((EBLK, 128), jnp.float32))
        scratch.append(pltpu.VMEM((64, 16), jnp.float32))
    scratch.append(pltpu.VMEM((80,), jnp.int32))
    scratch.append(pltpu.SemaphoreType.DMA)

    def body(*refs):
        i = 0
        if with_att:
            xl2, xr2, srcs, dsts, bounds, atts = refs[:6]
            osum, oden = refs[6:8]
            i = 8
        elif F > 0:
            xl2, srcs, dsts, bounds = refs[:4]
            osum, = refs[4:5]
            i = 5
        else:
            dsts, bounds = refs[:2]
            oden, = refs[2:3]
            i = 3
        r = list(refs[i:])
        accS = r.pop(0) if F > 0 else None
        accD = r.pop(0) if (with_att or F == 0) else None
        if F > 0:
            srcv, idxv, xlb = r.pop(0), r.pop(0), r.pop(0)
        dstv = r.pop(0)
        if with_att:
            xrb, attv = r.pop(0), r.pop(0)
        bvm, sem = r

        cc = lax.axis_index("c")
        ss = lax.axis_index("s")
        wid = ss * 2 + cc
        pltpu.sync_copy(bounds, bvm)
        iota = lax.iota(jnp.int32, 16)
        bvecs = [bvm[pl.ds(q * 16, 16)] for q in range(5)]

        def bscalar(k):
            i = k // 16
            j = k - i * 16
            sel = bvecs[0]
            for q in range(1, 5):
                sel = jnp.where(i == q, bvecs[q], sel)
            return _lane_take(sel, jnp.zeros((16,), jnp.int32) + j)[0]
        zf = jnp.zeros((16,), jnp.float32)

        def round_body(rnd, _):
            job = rnd * NW + wid
            h = job // NJOB
            chunk = job - h * NJOB
            d0 = chunk * CHK
            e0 = bscalar(chunk)
            e1 = bscalar(chunk + 1)
            if with_att:
                pltpu.sync_copy(atts.at[h], attv)

            if F > 0:
                def zs(j, _):
                    for q in range(F // 16):
                        accS[j, pl.ds(q * 16, 16)] = zf
                    return 0
                lax.fori_loop(0, CHK, zs, 0)
            if accD is not None:
                def zd(j, _):
                    accD[pl.ds(pl.multiple_of(j * 16, 16), 16)] = zf
                    return 0
                lax.fori_loop(0, CHK // 16, zd, 0)

            a0 = (e0 // EBLK) * EBLK
            nblk = (e1 - a0 + EBLK - 1) // EBLK

            def blk(b, _):
                base = pl.multiple_of(a0 + b * EBLK, EBLK)
                pltpu.sync_copy(dsts.at[pl.ds(base, EBLK)], dstv)
                if F > 0:
                    pltpu.sync_copy(srcs.at[pl.ds(base, EBLK)], srcv)
                    for g4 in range(EBLK // 16):
                        sl = pl.ds(g4 * 16, 16)
                        if H > 1:
                            idxv[sl] = srcv[sl] * H + h
                        else:
                            idxv[sl] = srcv[sl]
                    pltpu.async_copy(xl2.at[idxv], xlb, sem).wait()
                if with_att:
                    for g4 in range(EBLK // 16):
                        sl = pl.ds(g4 * 16, 16)
                        if H > 1:
                            idxv[sl] = dstv[sl] * H + h
                        else:
                            idxv[sl] = dstv[sl]
                    pltpu.async_copy(xr2.at[idxv], xrb, sem).wait()

                def grp(g, _):
                    goff = pl.multiple_of(g * 16, 16)
                    erow = g * 16 + iota
                    lanepos = base + g * 16 + iota
                    valid = (lanepos >= e0) & (lanepos < e1)
                    vmsk = jnp.where(valid, 1.0, 0.0)
                    dl = dstv[pl.ds(goff, 16)] - d0

                    if with_att:
                        acc = zf
                        for f in range(64):
                            fi = jnp.full((16,), f, jnp.int32)
                            xi = plsc.load_gather(xrb, [erow, fi])
                            xj = plsc.load_gather(xlb, [erow, fi])
                            sm = xi + xj
                            t = jnp.maximum(sm, 0.2 * sm)
                            acc = acc + t * attv[f, :]
                        cself = plsc.load_gather(
                            xrb, [erow, jnp.full((16,), 64, jnp.int32)])
                        ex = jnp.exp(acc - cself) * vmsk
                    else:
                        ex = vmsk

                    if accD is not None:
                        plsc.addupdate_scatter(accD, [dl], ex, mask=valid)
                    if F > 0:
                        for f in range(F):
                            fi = jnp.full((16,), f, jnp.int32)
                            xj = plsc.load_gather(xlb, [erow, fi])
                            plsc.addupdate_scatter(accS, [dl, fi], xj * ex,
                                                   mask=valid)
                    return 0

                lax.fori_loop(0, EBLK // 16, grp, 0)
                return 0

            lax.fori_loop(0, nblk, blk, 0)

            if with_att:
                ho = pl.multiple_of(h * NPAD + d0, 16)
                pltpu.sync_copy(accS, osum.at[pl.ds(ho, CHK), :])
                pltpu.sync_copy(accD, oden.at[pl.ds(ho, CHK)])
            elif F > 0:
                pltpu.sync_copy(accS, osum.at[pl.ds(d0, CHK), :])
            else:
                pltpu.sync_copy(accD, oden.at[pl.ds(d0, CHK)])
            return 0

        lax.fori_loop(0, rounds, round_body, 0)

    return pl.kernel(body, out_type=out_type, mesh=mesh,
                     scratch_types=scratch,
                     compiler_params=pltpu.CompilerParams(
                         needs_layout_passes=False))


def _sort_edges(ei):
    order = jnp.argsort(ei[1])
    src = jnp.take(ei[0], order)
    dst = jnp.take(ei[1], order)
    marks = (jnp.arange(NJOB + 1, dtype=jnp.int32) * CHK).astype(jnp.int32)
    bounds = jnp.searchsorted(dst, marks).astype(jnp.int32)
    bounds = jnp.concatenate(
        [bounds, jnp.full((80 - (NJOB + 1),), ei.shape[1], jnp.int32)])
    return src.astype(jnp.int32), dst.astype(jnp.int32), bounds


_B = 1000


def _full(s):
    return pl.BlockSpec(s, lambda i: (0,) * len(s))


def _rows(s):
    return pl.BlockSpec(s, lambda i: (i,) + (0,) * (len(s) - 1))


def _gru_step(x, h, ws):
    wrT, wzT, wnT, urT, uzT, unT, br, bz, bn_, cr, cz, cn = ws
    r = jax.nn.sigmoid(x @ wrT + br + h @ urT + cr)
    z = jax.nn.sigmoid(x @ wzT + bz + h @ uzT + cz)
    n = jnp.tanh(x @ wnT + bn_ + r * (h @ unT + cn))
    return (1.0 - z) * n + z * h


def _gru_dense_body(x_ref, *refs):
    w0 = [r[...] for r in refs[0:12]]
    w1 = [r[...] for r in refs[12:24]]
    dw = refs[24][...]
    db = refs[25][...]
    o_ref = refs[26]
    x = x_ref[...]
    h1 = jnp.zeros((x.shape[0], 25), jnp.float32)
    h2 = jnp.zeros((x.shape[0], 25), jnp.float32)
    enc = jnp.zeros((x.shape[0], 128), jnp.float32) + db
    for t in range(25):
        h1 = _gru_step(x[:, t, :], h1, w0)
        h2 = _gru_step(h1, h2, w1)
        enc = enc + h2 @ dw[t]
    o_ref[...] = jnp.maximum(enc, 0.0)


def _gru_dense(x_low, params):
    def unpack(p):
        ws = []
        for m in ('wih', 'whh'):
            w = p[m]
            ws += [w[0:25].T, w[25:50].T, w[50:75].T]
        for m in ('bih', 'bhh'):
            b = p[m]
            ws += [b[0:25], b[25:50], b[50:75]]
        return [ws[0], ws[1], ws[2], ws[3], ws[4], ws[5],
                ws[6], ws[7], ws[8], ws[9], ws[10], ws[11]]
    w0 = unpack(params['gru'][0])
    w1 = unpack(params['gru'][1])
    dw = params['dense']['w'].T.reshape(25, 25, 128)
    db = params['dense']['b']
    args = [x_low] + w0 + w1 + [dw, db]
    BG = 400
    specs = [_rows((BG, 25, 25))] + [_full(a.shape) for a in args[1:]]
    return pl.pallas_call(
        _gru_dense_body,
        grid=(N // BG,),
        in_specs=specs,
        out_specs=_rows((BG, 128)),
        out_shape=jax.ShapeDtypeStruct((N, 128), jnp.float32),
    )(*args)


def _down_epi_body(s_ref, c_ref, xh_ref, wrel_ref, brel_ref, wroot_ref,
                   y_ref, ps_ref, pq_ref):
    agg = s_ref[...] / jnp.maximum(c_ref[...], 1.0)
    y = agg @ wrel_ref[...] + brel_ref[...] + xh_ref[...] @ wroot_ref[...]
    y_ref[...] = y
    ps_ref[0, 0, :] = jnp.sum(y, axis=0)
    pq_ref[0, 0, :] = jnp.sum(y * y, axis=0)


def _down_epi(sumL, cntL, x_high, params):
    wrel = params['down']['w_rel'].T
    wroot = params['down']['w_root'].T
    brel = params['down']['b_rel']
    g = N // _B
    return pl.pallas_call(
        _down_epi_body,
        grid=(g,),
        in_specs=[_rows((_B, 128)), _rows((_B, 1)), _rows((_B, 7)),
                  _full(wrel.shape), _full(brel.shape), _full(wroot.shape)],
        out_specs=[_rows((_B, 64)), _rows((1, 1, 64)), _rows((1, 1, 64))],
        out_shape=[jax.ShapeDtypeStruct((N, 64), jnp.float32),
                   jax.ShapeDtypeStruct((g, 1, 64), jnp.float32),
                   jax.ShapeDtypeStruct((g, 1, 64), jnp.float32)],
    )(sumL, cntL, x_high, wrel, brel, wroot)


def _node_linear_body(relu, H, y_ref, sc_ref, sh_ref, wl_ref, wr_ref,
                      attP_ref, oxl_ref, oxr_ref):
    xn = y_ref[...] * sc_ref[...] + sh_ref[...]
    if relu:
        xn = jnp.maximum(xn, 0.0)
    xl = xn @ wl_ref[...]
    xr = xn @ wr_ref[...]
    for h in range(H):
        xl_h = xl[:, h * 64:(h + 1) * 64]
        xr_h = xr[:, h * 64:(h + 1) * 64]
        sm = xl_h + xr_h
        t = jnp.maximum(sm, 0.2 * sm)
        cp = t @ attP_ref[h]
        z = jnp.zeros((xl_h.shape[0], 64), jnp.float32)
        oxl_ref[:, h, :] = jnp.concatenate([xl_h, z], axis=-1)
        oxr_ref[:, h, :] = jnp.concatenate([xr_h, cp, z[:, :48]], axis=-1)


def _node_linear(y, scale, shift, gp, H, relu):
    fin = y.shape[1]
    wl = gp['wl'].T
    wr = gp['wr'].T
    attP = jnp.zeros((H, 64, 16), jnp.float32).at[:, :, 0].set(gp['att'])
    return pl.pallas_call(
        functools.partial(_node_linear_body, relu, H),
        grid=(N // _B,),
        in_specs=[_rows((_B, fin)), _full((fin,)), _full((fin,)),
                  _full(wl.shape), _full(wr.shape), _full(attP.shape)],
        out_specs=[_rows((_B, H, 128)), _rows((_B, H, 128))],
        out_shape=[jax.ShapeDtypeStruct((N, H, 128), jnp.float32),
                   jax.ShapeDtypeStruct((N, H, 128), jnp.float32)],
    )(y, scale, shift, wl, wr, attP)


def _gat_epi_body(H, last, pred, s_ref, d_ref, xl_ref, c_ref, b_ref, *orefs):
    cnt1 = c_ref[...] + 1.0
    ys = []
    for h in range(H):
        num = s_ref[:, h, :] + xl_ref[:, h, :64]
        den = d_ref[:, h:h + 1] + (1.0 + 1e-16)
        ys.append(num / den / cnt1)
    y = (jnp.concatenate(ys, axis=-1) if H > 1 else ys[0]) + b_ref[...]
    if not last:
        y_ref, ps_ref, pq_ref = orefs
        y_ref[...] = y
        ps_ref[0, 0, :] = jnp.sum(y, axis=0)
        pq_ref[0, 0, :] = jnp.sum(y * y, axis=0)
    else:
        w1, b1, w2, b2, w3, b3 = pred
        o_ref, = orefs
        y = jnp.maximum(y, 0.0)
        h1 = jnp.maximum(y @ w1[...] + b1[...], 0.0)
        h2 = jnp.maximum(h1 @ w2[...] + b2[...], 0.0)
        o_ref[...] = h2 @ w3[...] + b3[...]


def _gat_epi(gsum, gden, xl, cnt, bias, H, last=False, pred=None):
    F = H * 64
    g = N // _B
    ins = [gsum, gden, xl, cnt, bias]
    specs = [_rows((_B, H, 64)), _rows((_B, H)), _rows((_B, H, 128)),
             _rows((_B, 1)), _full((F,))]
    if not last:
        body = functools.partial(_gat_epi_body, H, False, None)
        outs = [jax.ShapeDtypeStruct((N, F), jnp.float32),
                jax.ShapeDtypeStruct((g, 1, F), jnp.float32),
                jax.ShapeDtypeStruct((g, 1, F), jnp.float32)]
        ospecs = [_rows((_B, F)), _rows((1, 1, F)), _rows((1, 1, F))]
        return pl.pallas_call(body, grid=(g,), in_specs=specs,
                              out_specs=ospecs, out_shape=outs)(*ins)
    pr = pred
    w3p = jnp.zeros((32, 128), jnp.float32).at[:, 0].set(pr['w3'][0])
    b3p = jnp.zeros((128,), jnp.float32).at[0].set(pr['b3'][0])
    pw = [pr['w1'].T, pr['b1'], pr['w2'].T, pr['b2'], w3p, b3p]
    nin = len(ins)

    def body(*refs):
        _gat_epi_body(H, True, refs[nin:nin + 6], *refs[:nin], *refs[nin + 6:])

    return pl.pallas_call(
        body, grid=(g,),
        in_specs=specs + [_full(a.shape) for a in pw],
        out_specs=[_rows((_B, 128))],
        out_shape=[jax.ShapeDtypeStruct((N, 128), jnp.float32)],
    )(*(ins + pw))


def _bn_scale_shift(ps, pq, bnp):
    mu = jnp.sum(ps, axis=(0, 1)) / N
    var = jnp.sum(pq, axis=(0, 1)) / N - mu * mu
    scale = bnp['g'] / jnp.sqrt(var + BN_EPS)
    shift = bnp['b'] - mu * scale
    return scale, shift


def kernel(x_low, x_high, edge_index_l2h, edge_index_hh, params):
    E = edge_index_hh.shape[1]
    EL = edge_index_l2h.shape[1]

    srcL, dstL, boundsL = _sort_edges(edge_index_l2h)
    srcH, dstH, boundsH = _sort_edges(edge_index_hh)

    enc = _gru_dense(x_low, params)

    count_k = _make_sc_seg(E, 0, 1, False)
    cntH = count_k(dstH, boundsH)[0][:N]
    cntL = count_k(dstL, boundsL)[0][:N]

    down_k = _make_sc_seg(EL, 128, 1, False)
    sumL = down_k(enc, srcL, dstL, boundsL)[0][:N]

    y, ps, pq = _down_epi(sumL, cntL.reshape(N, 1), x_high, params)
    scale, shift = _bn_scale_shift(ps, pq, params['bn'][0])

    gat2_k = _make_sc_seg(E, 64, 2, True)
    gat1_k = _make_sc_seg(E, 64, 1, True)
    cntH1 = cntH.reshape(N, 1)

    heads = [2, 2, 2, 2, 1]
    for i in range(5):
        H = heads[i]
        gp = params['gat'][i]
        relu = i > 0
        xl, xr = _node_linear(y, scale, shift, gp, H, relu)
        atts = jnp.broadcast_to(gp['att'][:, :, None], (H, 64, 16))
        kk = gat2_k if H == 2 else gat1_k
        osum, oden = kk(xl.reshape(N * H, 128), xr.reshape(N * H, 128),
                        srcH, dstH, boundsH, atts)
        osum = osum.reshape(H, NPAD, 64)
        oden = oden.reshape(H, NPAD)
        gsum = jnp.transpose(osum[:, :N, :], (1, 0, 2))
        gden = jnp.transpose(oden[:, :N], (1, 0))
        if i < 4:
            y, ps, pq = _gat_epi(gsum, gden, xl, cntH1, gp['bias'], H)
            scale, shift = _bn_scale_shift(ps, pq, params['bn'][i + 1])
        else:
            out = _gat_epi(gsum, gden, xl, cntH1, gp['bias'], H,
                           last=True, pred=params['pred'])[0]
    return out[:, :1]
